# baseline (speedup 1.0000x reference)
import functools

import jax
import jax.numpy as jnp
from jax import lax
from jax.experimental import pallas as pl
from jax.experimental.pallas import tpu as pltpu
from jax.experimental.pallas import tpu_sc as plsc

_PS = 16
_D = 768
_N = 576

_SC_CORES = 2
_ROWS_PER_WORKER = 8


def _sc_gather(y, table):
    B = y.shape[0]
    D = table.shape[1]
    nworkers = B // _ROWS_PER_WORKER
    mesh = plsc.VectorSubcoreMesh(core_axis_name="c", subcore_axis_name="s")

    @functools.partial(
        pl.kernel,
        mesh=mesh,
        out_type=jax.ShapeDtypeStruct((B, D), jnp.float32),
        scratch_types=[
            pltpu.VMEM((_ROWS_PER_WORKER,), jnp.int32),
            pltpu.VMEM((_ROWS_PER_WORKER, D), jnp.float32),
            pltpu.SemaphoreType.DMA,
        ],
    )
    def gather_kernel(y_hbm, table_hbm, out_hbm, idx_v, rows_v, sem):
        wid = lax.axis_index("s") * _SC_CORES + lax.axis_index("c")

        @pl.when(wid < nworkers)
        def _():
            base = wid * _ROWS_PER_WORKER
            pltpu.sync_copy(y_hbm.at[pl.ds(base, _ROWS_PER_WORKER)], idx_v)
            pltpu.async_copy(table_hbm.at[idx_v], rows_v, sem).wait()
            pltpu.sync_copy(rows_v, out_hbm.at[pl.ds(base, _ROWS_PER_WORKER)])

    return gather_kernel(y, table)


def _tc_body(cls_ref, x_ref, o_ref):
    xb = x_ref[0]
    xb = xb.reshape(3, _N // 24, _PS, _N // 24, _PS)
    px = jnp.transpose(xb, (1, 3, 0, 2, 4)).reshape(_N, _D)
    o_ref[0] = jnp.concatenate([cls_ref[0], px], axis=0)


def kernel(x, y, label_bank):
    B, C, H, W = x.shape
    table = label_bank.reshape(label_bank.shape[0], _D)
    cls = _sc_gather(y, table)
    cls = cls.reshape(B, 1, _D)
    out = pl.pallas_call(
        _tc_body,
        grid=(B,),
        in_specs=[
            pl.BlockSpec((1, 1, _D), lambda b: (b, 0, 0)),
            pl.BlockSpec((1, C, H, W), lambda b: (b, 0, 0, 0)),
        ],
        out_specs=pl.BlockSpec((1, 1 + _N, _D), lambda b: (b, 0, 0)),
        out_shape=jax.ShapeDtypeStruct((B, 1 + _N, _D), jnp.float32),
    )(cls, x)
    return out


# speedup vs baseline: 1.7685x; 1.7685x over previous
import functools

import jax
import jax.numpy as jnp
from jax import lax
from jax.experimental import pallas as pl
from jax.experimental.pallas import tpu as pltpu
from jax.experimental.pallas import tpu_sc as plsc

_PS = 16
_D = 768
_N = 576
_NP = 24

_SC_CORES = 2
_NW = 32
_CLS_ROWS = 8


def _sc_kernel(x5, y, table, B):
    mesh = plsc.VectorSubcoreMesh(core_axis_name="c", subcore_axis_name="s")
    tasks_per_worker = (B * _NP) // _NW

    @functools.partial(
        pl.kernel,
        mesh=mesh,
        out_type=jax.ShapeDtypeStruct((B, 1 + _N, _D), jnp.float32),
        scratch_types=[
            pltpu.VMEM((_CLS_ROWS,), jnp.int32),
            pltpu.VMEM((_CLS_ROWS, _D), jnp.float32),
            pltpu.VMEM((3, _PS, _NP * _PS), jnp.float32),
            pltpu.VMEM((_NP, _D), jnp.float32),
            pltpu.SemaphoreType.DMA,
        ],
    )
    def body(x_hbm, y_hbm, table_hbm, out_hbm, idx_v, rows_v, buf, buf2, sem):
        wid = lax.axis_index("s") * _SC_CORES + lax.axis_index("c")

        @pl.when(wid < B // _CLS_ROWS)
        def _():
            base = wid * _CLS_ROWS
            pltpu.sync_copy(y_hbm.at[pl.ds(base, _CLS_ROWS)], idx_v)
            pltpu.async_copy(table_hbm.at[idx_v], rows_v, sem).wait()
            for k in range(_CLS_ROWS):
                pltpu.sync_copy(rows_v.at[k], out_hbm.at[base + k, 0, :])

        def task(t, carry):
            task_id = wid * tasks_per_worker + t
            b = task_id // _NP
            ph = task_id % _NP
            pltpu.sync_copy(x_hbm.at[b, :, ph, :, :], buf)

            def qstep(q, c2):
                for c in range(3):
                    for i in range(_PS):
                        buf2[q, pl.ds(c * 256 + i * _PS, _PS)] = (
                            buf[c, i, pl.ds(q * _PS, _PS)])
                return c2
            lax.fori_loop(0, _NP, qstep, 0)

            descs = [
                pltpu.async_copy(
                    buf2.at[q], out_hbm.at[b, 1 + ph * _NP + q, :], sem)
                for q in range(_NP)
            ]
            for d in descs:
                d.wait()
            return carry
        lax.fori_loop(0, tasks_per_worker, task, 0)

    return body(x5, y, table)


def kernel(x, y, label_bank):
    B, C, H, W = x.shape
    table = label_bank.reshape(label_bank.shape[0], _D)
    x5 = x.reshape(B, C, _NP, _PS, W)
    return _sc_kernel(x5, y, table, B)


# speedup vs baseline: 2.2493x; 1.2719x over previous
import functools

import jax
import jax.numpy as jnp
from jax import lax
from jax.experimental import pallas as pl
from jax.experimental.pallas import tpu as pltpu
from jax.experimental.pallas import tpu_sc as plsc

_PS = 16
_D = 768
_N = 576
_NP = 24

_SC_CORES = 2
_NW = 32
_CLS_ROWS = 8


def _sc_kernel(x5, y, table, B):
    mesh = plsc.VectorSubcoreMesh(core_axis_name="c", subcore_axis_name="s")
    tpw = (B * _NP) // _NW

    @functools.partial(
        pl.kernel,
        mesh=mesh,
        out_type=jax.ShapeDtypeStruct((B, 1 + _N, _D), jnp.float32),
        scratch_types=[
            pltpu.VMEM((_CLS_ROWS,), jnp.int32),
            pltpu.VMEM((_CLS_ROWS, _D), jnp.float32),
            pltpu.VMEM((2, 3, _PS, _NP * _PS), jnp.float32),
            pltpu.VMEM((2, _NP, _D), jnp.float32),
            pltpu.SemaphoreType.DMA,
            pltpu.SemaphoreType.DMA,
            pltpu.SemaphoreType.DMA,
            pltpu.SemaphoreType.DMA,
        ],
    )
    def body(x_hbm, y_hbm, table_hbm, out_hbm, idx_v, rows_v, buf, buf2,
             gsem0, gsem1, ssem0, ssem1):
        wid = lax.axis_index("s") * _SC_CORES + lax.axis_index("c")
        gsems = (gsem0, gsem1)
        ssems = (ssem0, ssem1)

        @pl.when(wid < B // _CLS_ROWS)
        def _():
            base = wid * _CLS_ROWS
            pltpu.sync_copy(y_hbm.at[pl.ds(base, _CLS_ROWS)], idx_v)
            pltpu.async_copy(table_hbm.at[idx_v], rows_v, gsem0).wait()
            for k in range(_CLS_ROWS):
                pltpu.sync_copy(rows_v.at[k], out_hbm.at[base + k, 0, :])

        def bph(t):
            task_id = wid * tpw + t
            return task_id // _NP, task_id % _NP

        def gather(t, slot):
            b, ph = bph(t)
            return pltpu.make_async_copy(
                x_hbm.at[b, :, ph, :, :], buf.at[slot], gsems[slot])

        def relayout(slot):
            def qstep(q, c2):
                for c in range(3):
                    for i in range(_PS):
                        buf2[slot, q, pl.ds(c * 256 + i * _PS, _PS)] = (
                            buf[slot, c, i, pl.ds(q * _PS, _PS)])
                return c2
            lax.fori_loop(0, _NP, qstep, 0)

        def scatter_rows(t, slot):
            b, ph = bph(t)
            for q in range(_NP):
                pltpu.make_async_copy(
                    buf2.at[slot, q],
                    out_hbm.at[b, 1 + ph * _NP + q, :],
                    ssems[slot]).start()

        def drain_scatters(slot):
            pltpu.make_async_copy(
                out_hbm.at[0, pl.ds(8, _NP), :], buf2.at[slot],
                ssems[slot]).wait()

        def process(t, slot, tt):
            @pl.when(t + 1 < tpw)
            def _():
                gather(t + 1, 1 - slot).start()
            gather(t, slot).wait()
            @pl.when(tt >= 1)
            def _():
                drain_scatters(slot)
            relayout(slot)
            scatter_rows(t, slot)

        gather(0, 0).start()

        def step(tt, carry):
            process(2 * tt, 0, tt)
            process(2 * tt + 1, 1, tt)
            return carry
        lax.fori_loop(0, tpw // 2, step, 0)

        drain_scatters(0)
        drain_scatters(1)

    return body(x5, y, table)


def kernel(x, y, label_bank):
    B, C, H, W = x.shape
    table = label_bank.reshape(label_bank.shape[0], _D)
    x5 = x.reshape(B, C, _NP, _PS, W)
    return _sc_kernel(x5, y, table, B)


# speedup vs baseline: 3.2833x; 1.4597x over previous
import functools

import jax
import jax.numpy as jnp
from jax import lax
from jax.experimental import pallas as pl
from jax.experimental.pallas import tpu as pltpu
from jax.experimental.pallas import tpu_sc as plsc

_PS = 16
_D = 768
_N = 576
_NP = 24

_SC_CORES = 2
_NW = 32
_CLS_ROWS = 8


def _sc_kernel(x5, y, table, B):
    mesh = plsc.VectorSubcoreMesh(core_axis_name="c", subcore_axis_name="s")
    tpw = (B * _NP) // _NW

    @functools.partial(
        pl.kernel,
        mesh=mesh,
        out_type=jax.ShapeDtypeStruct((B, 1 + _N, _D), jnp.float32),
        scratch_types=[
            pltpu.VMEM((_CLS_ROWS,), jnp.int32),
            pltpu.VMEM((_CLS_ROWS, _D), jnp.float32),
            pltpu.VMEM((2, 3, _PS, _NP * _PS), jnp.float32),
            pltpu.VMEM((2, _NP, _D), jnp.float32),
            pltpu.SemaphoreType.DMA,
            pltpu.SemaphoreType.DMA,
            pltpu.SemaphoreType.DMA,
            pltpu.SemaphoreType.DMA,
        ],
    )
    def body(x_hbm, y_hbm, table_hbm, out_hbm, idx_v, rows_v, buf, buf2,
             gsem0, gsem1, ssem0, ssem1):
        wid = lax.axis_index("s") * _SC_CORES + lax.axis_index("c")
        gsems = (gsem0, gsem1)
        ssems = (ssem0, ssem1)

        @pl.when(wid < B // _CLS_ROWS)
        def _():
            base = wid * _CLS_ROWS
            pltpu.sync_copy(y_hbm.at[pl.ds(base, _CLS_ROWS)], idx_v)
            pltpu.async_copy(table_hbm.at[idx_v], rows_v, gsem0).wait()
            for k in range(_CLS_ROWS):
                pltpu.sync_copy(rows_v.at[k], out_hbm.at[base + k, 0, :])

        def bph(t):
            task_id = wid * tpw + t
            return task_id // _NP, task_id % _NP

        def gather(t, slot):
            b, ph = bph(t)
            return pltpu.make_async_copy(
                x_hbm.at[b, :, ph, :, :], buf.at[slot], gsems[slot])

        def relayout(slot):
            for q in range(_NP):
                for c in range(3):
                    for i in range(_PS):
                        buf2[slot, q, pl.ds(c * 256 + i * _PS, _PS)] = (
                            buf[slot, c, i, pl.ds(q * _PS, _PS)])

        def scatter_rows(t, slot):
            b, ph = bph(t)
            for q in range(_NP):
                pltpu.make_async_copy(
                    buf2.at[slot, q],
                    out_hbm.at[b, 1 + ph * _NP + q, :],
                    ssems[slot]).start()

        def drain_scatters(slot):
            pltpu.make_async_copy(
                out_hbm.at[0, pl.ds(8, _NP), :], buf2.at[slot],
                ssems[slot]).wait()

        def process(t, slot, tt):
            @pl.when(t + 1 < tpw)
            def _():
                gather(t + 1, 1 - slot).start()
            gather(t, slot).wait()
            @pl.when(tt >= 1)
            def _():
                drain_scatters(slot)
            relayout(slot)
            scatter_rows(t, slot)

        gather(0, 0).start()

        def step(tt, carry):
            process(2 * tt, 0, tt)
            process(2 * tt + 1, 1, tt)
            return carry
        lax.fori_loop(0, tpw // 2, step, 0)

        drain_scatters(0)
        drain_scatters(1)

    return body(x5, y, table)


def kernel(x, y, label_bank):
    B, C, H, W = x.shape
    table = label_bank.reshape(label_bank.shape[0], _D)
    x5 = x.reshape(B, C, _NP, _PS, W)
    return _sc_kernel(x5, y, table, B)
